# Initial kernel scaffold; baseline (speedup 1.0000x reference)
import functools

import jax
import jax.numpy as jnp
from jax import lax
from jax.experimental import pallas as pl
from jax.experimental.pallas import tpu as pltpu
from jax.experimental.pallas import tpu_sc as plsc

_LENS = [44, 7, 6, 7, 2, 2, 6, 7]
_IN0 = sum(_LENS)
_H = 128
_N = 10000
_NE = 320000
_NG = 256
_NW = 32
_C = 128
_K = 80
_NEP = _NW * _K * _C
_NSEG = 10016
_ZROWS = _NSEG // 16
_OROWS = _N // 16


def _sc_agg(h, srcp, dstp, ea0p, ea1p, web, zer):
    mesh = plsc.VectorSubcoreMesh(core_axis_name="c", subcore_axis_name="s")

    @functools.partial(
        pl.kernel,
        mesh=mesh,
        out_type=jax.ShapeDtypeStruct((2, _N, _H), jnp.float32),
        scratch_types=[
            pltpu.VMEM((_C,), jnp.int32),
            pltpu.VMEM((_C,), jnp.int32),
            pltpu.VMEM((_C,), jnp.float32),
            pltpu.VMEM((_C,), jnp.float32),
            pltpu.VMEM((_C, _H), jnp.float32),
            pltpu.VMEM((3, _H), jnp.float32),
            pltpu.VMEM_SHARED((_NSEG, _H), jnp.float32),
            pltpu.SemaphoreType.DMA,
        ],
    )
    def k(h_hbm, src_hbm, dst_hbm, e0_hbm, e1_hbm, web_hbm, zer_hbm, out_hbm,
          siv, div, e0v, e1v, rows, webv, agg, sem):
        cid = lax.axis_index("c")
        sid = lax.axis_index("s")
        wid = sid * 2 + cid
        pltpu.sync_copy(zer_hbm, agg.at[pl.ds(sid * _ZROWS, _ZROWS)])
        pltpu.sync_copy(web_hbm, webv)
        plsc.subcore_barrier()

        w0 = tuple(webv[0, pl.ds(16 * f, 16)] for f in range(8))
        w1 = tuple(webv[1, pl.ds(16 * f, 16)] for f in range(8))
        bb = tuple(webv[2, pl.ds(16 * f, 16)] for f in range(8))

        def chunk_body(kk, carry):
            cw0, cw1, cbb = carry
            pltpu.sync_copy(src_hbm.at[wid, kk], siv)
            pltpu.sync_copy(dst_hbm.at[wid, kk], div)
            pltpu.sync_copy(e0_hbm.at[wid, kk], e0v)
            pltpu.sync_copy(e1_hbm.at[wid, kk], e1v)
            pltpu.async_copy(h_hbm.at[siv], rows, sem).wait()

            def edge_body(c, cc):
                ew0, ew1, ebb = cc
                s0 = e0v[c]
                s1 = e1v[c]
                for f in range(8):
                    v = rows[c, pl.ds(16 * f, 16)]
                    v = jnp.maximum(v + s0 * ew0[f] + s1 * ew1[f] + ebb[f], 0.0)
                    rows[c, pl.ds(16 * f, 16)] = v
                return cc

            lax.fori_loop(0, _C, edge_body, (cw0, cw1, cbb))
            pltpu.sync_copy(rows, agg.at[div], add=True)
            return carry

        lax.fori_loop(0, _K, chunk_body, (w0, w1, bb))
        plsc.subcore_barrier()
        pltpu.sync_copy(agg.at[pl.ds(sid * _OROWS, _OROWS)],
                        out_hbm.at[cid, pl.ds(sid * _OROWS, _OROWS)])

    return k(h, srcp, dstp, ea0p, ea1p, web, zer)


def _embed_body(xt_ref, e_ref, out_ref):
    iot = lax.broadcasted_iota(jnp.int32, (_H, _N), 0)
    acc = jnp.zeros((_H, _N), jnp.float32)
    off = 0
    for i, s in enumerate(_LENS):
        acc += (iot == xt_ref[i:i + 1, :] + off).astype(jnp.float32)
        off += s
    out_ref[...] = lax.dot_general(acc, e_ref[...], (((0,), (0,)), ((), ())),
                                   preferred_element_type=jnp.float32)


def _embed(xt, emat):
    return pl.pallas_call(
        _embed_body,
        out_shape=jax.ShapeDtypeStruct((_N, _H), jnp.float32),
    )(xt, emat)


def _bn(u, g, b):
    mu = jnp.mean(u, axis=0, keepdims=True)
    va = jnp.mean((u - mu) ** 2, axis=0, keepdims=True)
    return g * (u - mu) / jnp.sqrt(va + 1e-5) + b


def _leaky(u):
    return jnp.where(u >= 0, u, 0.01 * u)


def _dense_body(h_ref, a_ref, eps_ref, w1_ref, b1_ref, g1_ref, t1_ref,
                w2_ref, b2_ref, g2_ref, t2_ref, gn_ref, tn_ref, out_ref):
    z = h_ref[...] * (1.0 + eps_ref[0, 0]) + a_ref[0] + a_ref[1]
    u = jnp.dot(z, w1_ref[...], preferred_element_type=jnp.float32) + b1_ref[...]
    u = _leaky(_bn(u, g1_ref[...], t1_ref[...]))
    u = jnp.dot(u, w2_ref[...], preferred_element_type=jnp.float32) + b2_ref[...]
    u = _leaky(_bn(u, g2_ref[...], t2_ref[...]))
    out_ref[...] = _leaky(_bn(u, gn_ref[...], tn_ref[...]))


def _dense(h, agg2, eps, w1, b1, g1, t1, w2, b2, g2, t2, gn, tn):
    return pl.pallas_call(
        _dense_body,
        out_shape=jax.ShapeDtypeStruct((_N, _H), jnp.float32),
    )(h, agg2, eps, w1, b1, g1, t1, w2, b2, g2, t2, gn, tn)


def _pool_body(h_ref, b_ref, w1_ref, b1_ref, g1_ref, t1_ref,
               w2_ref, b2_ref, g2_ref, t2_ref, w3_ref, b3_ref, out_ref):
    iot = lax.broadcasted_iota(jnp.int32, (_NG, _N), 0)
    p = (iot == b_ref[0:1, :]).astype(jnp.float32)
    g = jnp.dot(p, h_ref[...], preferred_element_type=jnp.float32)
    u = jnp.dot(g, w1_ref[...], preferred_element_type=jnp.float32) + b1_ref[...]
    u = jnp.maximum(_bn(u, g1_ref[...], t1_ref[...]), 0.0)
    u = jnp.dot(u, w2_ref[...], preferred_element_type=jnp.float32) + b2_ref[...]
    u = jnp.maximum(_bn(u, g2_ref[...], t2_ref[...]), 0.0)
    out_ref[...] = jnp.dot(u, w3_ref[...], preferred_element_type=jnp.float32) + b3_ref[...]


def _pool(h, bt, w1, b1, g1, t1, w2, b2, g2, t2, w3, b3):
    return pl.pallas_call(
        _pool_body,
        out_shape=jax.ShapeDtypeStruct((_NG, _H), jnp.float32),
    )(h, bt, w1, b1, g1, t1, w2, b2, g2, t2, w3, b3)


def _padc(a, n):
    pad = [(0, 0)] * (a.ndim - 1) + [(0, n - a.shape[-1])]
    return jnp.pad(a.astype(jnp.float32), pad)


def kernel(x, edge_index, edge_attr, batch_idx, params):
    f32 = jnp.float32
    emat = jnp.zeros((_H, _H), f32)
    off = 0
    for i, s in enumerate(_LENS):
        emat = emat.at[off:off + s, off:off + s].set(params['emb'][i].astype(f32))
        off += s
    xt = x.T.astype(jnp.int32)

    npad = _NEP - _NE
    src = jnp.concatenate([edge_index[0].astype(jnp.int32),
                           jnp.zeros((npad,), jnp.int32)]).reshape(_NW, _K, _C)
    trash = _N + (jnp.arange(npad, dtype=jnp.int32) % 16)
    dst = jnp.concatenate([edge_index[1].astype(jnp.int32),
                           trash]).reshape(_NW, _K, _C)
    ea0 = jnp.concatenate([edge_attr[:, 0].astype(f32),
                           jnp.zeros((npad,), f32)]).reshape(_NW, _K, _C)
    ea1 = jnp.concatenate([edge_attr[:, 1].astype(f32),
                           jnp.zeros((npad,), f32)]).reshape(_NW, _K, _C)
    zer = jnp.zeros((_ZROWS, _H), f32)

    h = _embed(xt, emat)
    for p in params['layers']:
        web = jnp.concatenate([_padc(p['We'], _H),
                               _padc(p['be'][None, :], _H)], axis=0)
        agg2 = _sc_agg(h, src, dst, ea0, ea1, web, zer)
        w1 = jnp.pad(p['W1'].astype(f32), ((0, _H - p['W1'].shape[0]), (0, 0)))
        eps = p['eps'].astype(f32).reshape(1, 1)
        h = _dense(h, agg2, eps,
                   w1, p['b1'].reshape(1, _H), p['g1'].reshape(1, _H),
                   p['bt1'].reshape(1, _H),
                   p['W2'].astype(f32), p['b2'].reshape(1, _H),
                   p['g2'].reshape(1, _H), p['bt2'].reshape(1, _H),
                   p['gn'].reshape(1, _H), p['btn'].reshape(1, _H))

    bt = jnp.broadcast_to(batch_idx.astype(jnp.int32)[None, :], (8, _N))
    mp = params['mlp']
    h2 = mp['W1'].shape[1]
    out = _pool(h, bt,
                _padc(mp['W1'], _H), _padc(mp['b1'][None, :], _H),
                _padc(mp['g1'][None, :], _H), _padc(mp['bt1'][None, :], _H),
                _padc(jnp.pad(mp['W2'].astype(f32), ((0, _H - h2), (0, 0))), _H),
                _padc(mp['b2'][None, :], _H),
                _padc(mp['g2'][None, :], _H), _padc(mp['bt2'][None, :], _H),
                _padc(jnp.pad(mp['W3'].astype(f32), ((0, _H - h2), (0, 0))), _H),
                _padc(mp['b3'][None, :], _H))
    return out[:, :2]


# speedup vs baseline: 1.3281x; 1.3281x over previous
import functools

import jax
import jax.numpy as jnp
from jax import lax
from jax.experimental import pallas as pl
from jax.experimental.pallas import tpu as pltpu
from jax.experimental.pallas import tpu_sc as plsc

_LENS = [44, 7, 6, 7, 2, 2, 6, 7]
_IN0 = sum(_LENS)
_H = 128
_N = 10000
_NE = 320000
_NG = 256
_NW = 32
_C = 128
_K = 80
_NCH = _NW * _K
_EXT = 2
_NCHE = _NCH + _EXT
_NEP = _NCH * _C
_NSEG = 10240
_TRASH = 224
_ZROWS = _NSEG // 16
_OROWS = _NSEG // 16


def _sc_agg(h, srcf, samef, scat1f, extscf, owns, e0f, e1f, web, zer):
    mesh = plsc.VectorSubcoreMesh(core_axis_name="c", subcore_axis_name="s")

    @functools.partial(
        pl.kernel,
        mesh=mesh,
        out_type=jax.ShapeDtypeStruct((2, _NSEG, _H), jnp.float32),
        scratch_types=[
            pltpu.VMEM((_C,), jnp.int32),
            pltpu.VMEM((_C,), jnp.int32),
            pltpu.VMEM((_C,), jnp.float32),
            pltpu.VMEM((_C,), jnp.float32),
            pltpu.VMEM((_C,), jnp.float32),
            pltpu.VMEM((_C, _H), jnp.float32),
            pltpu.VMEM((3, _H), jnp.float32),
            pltpu.VMEM((16,), jnp.int32),
            pltpu.VMEM_SHARED((_NSEG, _H), jnp.float32),
            pltpu.SemaphoreType.DMA,
        ],
    )
    def k(h_hbm, src_hbm, same_hbm, scat1_hbm, ext_hbm, owns_hbm, e0_hbm,
          e1_hbm, web_hbm, zer_hbm, out_hbm,
          siv, scv, smv, e0v, e1v, rows, webv, ownv, agg, sem):
        cid = lax.axis_index("c")
        sid = lax.axis_index("s")
        wid = sid * 2 + cid
        pltpu.sync_copy(zer_hbm, agg.at[pl.ds(sid * _ZROWS, _ZROWS)])
        pltpu.sync_copy(web_hbm, webv)
        pltpu.sync_copy(owns_hbm.at[wid], ownv)
        plsc.subcore_barrier()

        w0 = tuple(webv[0, pl.ds(16 * f, 16)] for f in range(8))
        w1 = tuple(webv[1, pl.ds(16 * f, 16)] for f in range(8))
        bb = tuple(webv[2, pl.ds(16 * f, 16)] for f in range(8))
        zero16 = jnp.zeros((16,), jnp.float32)

        owns_s = ownv[...][0]

        def process(g, scat_hbm, prev, is_ext=False):
            pltpu.sync_copy(src_hbm.at[g], siv)
            pltpu.sync_copy(same_hbm.at[g], smv)
            pltpu.sync_copy(scat_hbm.at[g], scv)
            pltpu.sync_copy(e0_hbm.at[g], e0v)
            pltpu.sync_copy(e1_hbm.at[g], e1v)
            pltpu.async_copy(h_hbm.at[siv], rows, sem).wait()
            if is_ext:
                for g2 in range(_C // 16):
                    sl = pl.ds(16 * g2, 16)
                    tv = _N + 16 * g2 + lax.iota(jnp.int32, 16)
                    scv[sl] = jnp.where(owns_s > 0, scv[sl], tv)

            def group(g2, pv):
                s0v = e0v[pl.ds(16 * g2, 16)]
                s1v = e1v[pl.ds(16 * g2, 16)]
                sm16 = smv[pl.ds(16 * g2, 16)]
                for j in range(16):
                    c = g2 * 16 + j
                    s0 = s0v[j]
                    s1 = s1v[j]
                    sm = sm16[j]
                    nxt = []
                    for f in range(8):
                        ev = (s0 * w0[f] + s1 * w1[f]) + bb[f]
                        v = rows[c, pl.ds(16 * f, 16)]
                        v = jnp.maximum(v + ev, 0.0) + sm * pv[f]
                        rows[c, pl.ds(16 * f, 16)] = v
                        nxt.append(v)
                    pv = tuple(nxt)
                return pv

            prev = lax.fori_loop(0, _C // 16, group, prev)
            pltpu.sync_copy(rows, agg.at[scv], add=True)
            return prev

        def own_body(kk, prev):
            return process(wid * _K + kk, scat1_hbm, prev)

        prev = lax.fori_loop(0, _K, own_body, (zero16,) * 8)

        for ee in range(_EXT):
            prev = process(wid * _K + _K + ee, ext_hbm, prev, is_ext=True)

        plsc.subcore_barrier()
        pltpu.sync_copy(agg.at[pl.ds(sid * _OROWS, _OROWS)],
                        out_hbm.at[cid, pl.ds(sid * _OROWS, _OROWS)])

    return k(h, srcf, samef, scat1f, extscf, owns, e0f, e1f, web, zer)


def _embed_body(xt_ref, e_ref, out_ref):
    iot = lax.broadcasted_iota(jnp.int32, (_H, _N), 0)
    acc = jnp.zeros((_H, _N), jnp.float32)
    off = 0
    for i, s in enumerate(_LENS):
        acc += (iot == xt_ref[i:i + 1, :] + off).astype(jnp.float32)
        off += s
    out_ref[...] = lax.dot_general(acc, e_ref[...], (((0,), (0,)), ((), ())),
                                   preferred_element_type=jnp.float32,
                                   precision=lax.Precision.HIGHEST)


def _embed(xt, emat):
    return pl.pallas_call(
        _embed_body,
        out_shape=jax.ShapeDtypeStruct((_N, _H), jnp.float32),
    )(xt, emat)


def _dotbf(a, b):
    return jnp.dot(a.astype(jnp.bfloat16), b.astype(jnp.bfloat16),
                   preferred_element_type=jnp.float32)


def _bn(u, g, b):
    mu = jnp.mean(u, axis=0)
    va = jnp.mean((u - mu) ** 2, axis=0)
    return g * (u - mu) / jnp.sqrt(va + 1e-5) + b


def _leaky(u):
    return jnp.where(u >= 0, u, 0.01 * u)


def _mm1_body(h_ref, a_ref, eps_ref, w_ref, b_ref, out_ref):
    z = h_ref[...] * (1.0 + eps_ref[0, 0]) + a_ref[0, :_N, :] + a_ref[1, :_N, :]
    out_ref[...] = _dotbf(z, w_ref[...]) + b_ref[...]


def _mm1(h, agg2, eps, w, b):
    return pl.pallas_call(
        _mm1_body,
        out_shape=jax.ShapeDtypeStruct((_N, _H), jnp.float32),
    )(h, agg2, eps, w, b)


def _mm2_body(u_ref, w_ref, b_ref, out_ref):
    out_ref[...] = _dotbf(u_ref[...], w_ref[...]) + b_ref[...]


def _mm2(u, w, b):
    return pl.pallas_call(
        _mm2_body,
        out_shape=jax.ShapeDtypeStruct((_N, _H), jnp.float32),
    )(u, w, b)


def _pool_body(h_ref, b_ref, w1_ref, b1_ref, g1_ref, t1_ref,
               w2_ref, b2_ref, g2_ref, t2_ref, w3_ref, b3_ref, out_ref):
    iot = lax.broadcasted_iota(jnp.int32, (_NG, _N), 0)
    p = (iot == b_ref[0:1, :]).astype(jnp.float32)
    g = jnp.dot(p, h_ref[...], preferred_element_type=jnp.float32,
                precision=lax.Precision.HIGHEST)
    u = _dotbf(g, w1_ref[...]) + b1_ref[...]
    u = jnp.maximum(_bn(u, g1_ref[...], t1_ref[...]), 0.0)
    u = _dotbf(u, w2_ref[...]) + b2_ref[...]
    u = jnp.maximum(_bn(u, g2_ref[...], t2_ref[...]), 0.0)
    out_ref[...] = _dotbf(u, w3_ref[...]) + b3_ref[...]


def _pool(h, bt, w1, b1, g1, t1, w2, b2, g2, t2, w3, b3):
    return pl.pallas_call(
        _pool_body,
        out_shape=jax.ShapeDtypeStruct((_NG, _H), jnp.float32),
    )(h, bt, w1, b1, g1, t1, w2, b2, g2, t2, w3, b3)


def _padc(a, n):
    pad = [(0, 0)] * (a.ndim - 1) + [(0, n - a.shape[-1])]
    return jnp.pad(a.astype(jnp.float32), pad)


def kernel(x, edge_index, edge_attr, batch_idx, params):
    f32 = jnp.float32
    emat = jnp.zeros((_H, _H), f32)
    off = 0
    for i, s in enumerate(_LENS):
        emat = emat.at[off:off + s, off:off + s].set(params['emb'][i].astype(f32))
        off += s
    xt = x.T.astype(jnp.int32)

    src0 = edge_index[0].astype(jnp.int32)
    dst0 = edge_index[1].astype(jnp.int32)
    eab = edge_attr.astype(jnp.bfloat16).astype(f32)
    order = jnp.argsort(dst0, stable=True)
    ss = src0[order]
    ds = dst0[order]
    e0s = eab[:, 0][order]
    e1s = eab[:, 1][order]
    nall = _NCHE * _C
    npad = nall - _NE
    ig = jnp.arange(nall, dtype=jnp.int32)
    trash = _N + (ig % _TRASH)
    dsp = jnp.concatenate([ds, trash[_NE:]])
    srcf = jnp.concatenate([ss, jnp.zeros((npad,), jnp.int32)])
    e0f = jnp.concatenate([e0s, jnp.zeros((npad,), f32)])
    e1f = jnp.concatenate([e1s, jnp.zeros((npad,), f32)])
    same = jnp.concatenate([jnp.zeros((1,), jnp.bool_), dsp[1:] == dsp[:-1]])
    run_end = jnp.concatenate([dsp[1:] != dsp[:-1], jnp.ones((1,), jnp.bool_)])
    scat = jnp.where(run_end, dsp, trash)
    lead2 = jnp.cumprod(same[:_NEP].reshape(_NW, -1).astype(jnp.int32), axis=1)
    lead = jnp.concatenate([lead2.reshape(-1) > 0,
                            jnp.zeros((_EXT * _C,), jnp.bool_)])
    scat1 = jnp.where(lead, trash, scat)[:_NEP].reshape(_NCH, _C)
    extsc = jnp.where(lead, scat, trash).reshape(_NCHE, _C)
    owns = jnp.broadcast_to((1 - lead2[:, -1])[:, None], (_NW, 16)).astype(jnp.int32)
    samef = same.astype(f32).reshape(_NCHE, _C)
    srcf = srcf.reshape(_NCHE, _C)
    e0f = e0f.reshape(_NCHE, _C)
    e1f = e1f.reshape(_NCHE, _C)
    zer = jnp.zeros((_ZROWS, _H), f32)

    h = _embed(xt, emat)
    for p in params['layers']:
        web = jnp.concatenate(
            [_padc(p['We'].astype(jnp.bfloat16).astype(f32), _H),
             _padc(p['be'][None, :], _H)], axis=0)
        agg2 = _sc_agg(h, srcf, samef, scat1, extsc, owns, e0f, e1f, web, zer)
        w1 = jnp.pad(p['W1'].astype(f32), ((0, _H - p['W1'].shape[0]), (0, 0)))
        eps = p['eps'].astype(f32).reshape(1, 1)
        u = _mm1(h, agg2, eps, w1, p['b1'].reshape(1, _H))
        u = _leaky(_bn(u, p['g1'], p['bt1']))
        u = _mm2(u, p['W2'].astype(f32), p['b2'].reshape(1, _H))
        u = _leaky(_bn(u, p['g2'], p['bt2']))
        h = _leaky(_bn(u, p['gn'], p['btn']))

    bt = jnp.broadcast_to(batch_idx.astype(jnp.int32)[None, :], (8, _N))
    mp = params['mlp']
    h2 = mp['W1'].shape[1]
    out = _pool(h, bt,
                _padc(mp['W1'], _H), _padc(mp['b1'][None, :], _H),
                _padc(mp['g1'][None, :], _H), _padc(mp['bt1'][None, :], _H),
                _padc(jnp.pad(mp['W2'].astype(f32), ((0, _H - h2), (0, 0))), _H),
                _padc(mp['b2'][None, :], _H),
                _padc(mp['g2'][None, :], _H), _padc(mp['bt2'][None, :], _H),
                _padc(jnp.pad(mp['W3'].astype(f32), ((0, _H - h2), (0, 0))), _H),
                _padc(mp['b3'][None, :], _H))
    return out[:, :2]
